# initial kernel scaffold (unmeasured)
import functools

import jax
import jax.numpy as jnp
from jax import lax
from jax.experimental import pallas as pl
from jax.experimental.pallas import tpu as pltpu

B, SQ, H, D = 8, 8, 16, 128
SKV_LOCAL = 1024
HALF = SKV_LOCAL // 2
SCALE = D ** -0.5


def kernel(Q, K, V):
    def body(q_ref, k_hbm, v_hbm, o_ref,
             k_buf, v_buf, o_acc, o_recv, st_acc, st_recv,
             copy_sems, send_sems, recv_sems):
        my_x = lax.axis_index("x")
        my_y = lax.axis_index("y")
        y_peer = (my_x, 1 - my_y)
        x_peer = (1 - my_x, my_y)

        barrier = pltpu.get_barrier_semaphore()
        for peer in (y_peer, x_peer):
            pl.semaphore_signal(barrier, inc=1, device_id=peer,
                                device_id_type=pl.DeviceIdType.MESH)
        pl.semaphore_wait(barrier, 2)

        row0 = my_x * HALF

        for b in range(B):
            kc = pltpu.make_async_copy(
                k_hbm.at[b, pl.ds(row0, HALF)], k_buf, copy_sems.at[0])
            vc = pltpu.make_async_copy(
                v_hbm.at[b, pl.ds(row0, HALF)], v_buf, copy_sems.at[1])
            kc.start()
            vc.start()
            kc.wait()
            vc.wait()
            q_b = q_ref[b]
            s = jnp.einsum("qhd,khd->hqk", q_b, k_buf[...],
                           preferred_element_type=jnp.float32) * SCALE
            m = jnp.max(s, axis=-1)
            p = jnp.exp(s - m[..., None])
            l = jnp.sum(p, axis=-1)
            o_b = jnp.einsum("hqk,khd->qhd", p, v_buf[...],
                             preferred_element_type=jnp.float32)
            o_acc[b] = o_b
            st_acc[0, b] = m.T
            st_acc[1, b] = l.T

        for phase, peer in enumerate((y_peer, x_peer)):
            o_rdma = pltpu.make_async_remote_copy(
                src_ref=o_acc, dst_ref=o_recv.at[phase],
                send_sem=send_sems.at[2 * phase],
                recv_sem=recv_sems.at[2 * phase],
                device_id=peer, device_id_type=pl.DeviceIdType.MESH)
            st_rdma = pltpu.make_async_remote_copy(
                src_ref=st_acc, dst_ref=st_recv.at[phase],
                send_sem=send_sems.at[2 * phase + 1],
                recv_sem=recv_sems.at[2 * phase + 1],
                device_id=peer, device_id_type=pl.DeviceIdType.MESH)
            o_rdma.start()
            st_rdma.start()
            o_rdma.wait()
            st_rdma.wait()

            m_s = st_acc[0]
            l_s = st_acc[1]
            m_p = st_recv[phase, 0]
            l_p = st_recv[phase, 1]
            m_n = jnp.maximum(m_s, m_p)
            a = jnp.exp(m_s - m_n)
            bt = jnp.exp(m_p - m_n)
            st_acc[0] = m_n
            st_acc[1] = a * l_s + bt * l_p
            o_acc[...] = (a[..., None] * o_acc[...]
                          + bt[..., None] * o_recv[phase])

        o_ref[...] = o_acc[...] / st_acc[1][..., None]

        @functools.partial(pl.run_scoped, sem=pltpu.SemaphoreType.REGULAR)
        def _(sem):
            for peer in (y_peer, x_peer):
                pl.semaphore_signal(sem, inc=1, device_id=peer,
                                    device_id_type=pl.DeviceIdType.MESH)
            pl.semaphore_wait(sem, 2)

    return pl.pallas_call(
        body,
        out_shape=jax.ShapeDtypeStruct((B, SQ, H, D), jnp.float32),
        in_specs=[
            pl.BlockSpec(memory_space=pltpu.VMEM),
            pl.BlockSpec(memory_space=pltpu.ANY),
            pl.BlockSpec(memory_space=pltpu.ANY),
        ],
        out_specs=pl.BlockSpec(memory_space=pltpu.VMEM),
        scratch_shapes=[
            pltpu.VMEM((HALF, H, D), jnp.float32),
            pltpu.VMEM((HALF, H, D), jnp.float32),
            pltpu.VMEM((B, SQ, H, D), jnp.float32),
            pltpu.VMEM((2, B, SQ, H, D), jnp.float32),
            pltpu.VMEM((2, B, SQ, H), jnp.float32),
            pltpu.VMEM((2, 2, B, SQ, H), jnp.float32),
            pltpu.SemaphoreType.DMA((2,)),
            pltpu.SemaphoreType.DMA((4,)),
            pltpu.SemaphoreType.DMA((4,)),
        ],
        compiler_params=pltpu.CompilerParams(collective_id=0),
    )(Q, K, V)


# baseline (device time: 98427 ns/iter reference)
import functools

import jax
import jax.numpy as jnp
from jax import lax
from jax.experimental import pallas as pl
from jax.experimental.pallas import tpu as pltpu

B, SQ, H, D = 8, 8, 16, 128
SKV_LOCAL = 1024
HALF = SKV_LOCAL // 2
SCALE = D ** -0.5


def kernel(Q, K, V):
    def body(q_ref, k_hbm, v_hbm, o_ref,
             k_buf, v_buf, o_acc, o_recv, st_acc, st_recv,
             copy_sems, send_sems, recv_sems):
        my_x = lax.axis_index("x")
        my_y = lax.axis_index("y")
        y_peer = (my_x, 1 - my_y)
        x_peer = (1 - my_x, my_y)

        barrier = pltpu.get_barrier_semaphore()
        for peer in (y_peer, x_peer):
            pl.semaphore_signal(barrier, inc=1, device_id=peer,
                                device_id_type=pl.DeviceIdType.MESH)
        pl.semaphore_wait(barrier, 2)

        row0 = my_x * HALF

        for b in range(B):
            kc = pltpu.make_async_copy(
                k_hbm.at[b, pl.ds(row0, HALF)], k_buf, copy_sems.at[0])
            vc = pltpu.make_async_copy(
                v_hbm.at[b, pl.ds(row0, HALF)], v_buf, copy_sems.at[1])
            kc.start()
            vc.start()
            kc.wait()
            vc.wait()
            ms = []
            ls = []
            for h in range(H):
                q_h = q_ref[b, :, h, :]
                k_h = k_buf[:, h, :]
                v_h = v_buf[:, h, :]
                s = lax.dot_general(
                    q_h, k_h, (((1,), (1,)), ((), ())),
                    preferred_element_type=jnp.float32) * SCALE
                m = jnp.max(s, axis=-1)
                p = jnp.exp(s - m[:, None])
                l = jnp.sum(p, axis=-1)
                o_h = lax.dot_general(
                    p, v_h, (((1,), (0,)), ((), ())),
                    preferred_element_type=jnp.float32)
                o_acc[b, :, h, :] = o_h
                ms.append(m)
                ls.append(l)
            st_acc[0, b] = jnp.stack(ms, axis=1)
            st_acc[1, b] = jnp.stack(ls, axis=1)

        for phase, peer in enumerate((y_peer, x_peer)):
            o_rdma = pltpu.make_async_remote_copy(
                src_ref=o_acc, dst_ref=o_recv.at[phase],
                send_sem=send_sems.at[2 * phase],
                recv_sem=recv_sems.at[2 * phase],
                device_id=peer, device_id_type=pl.DeviceIdType.MESH)
            st_rdma = pltpu.make_async_remote_copy(
                src_ref=st_acc, dst_ref=st_recv.at[phase],
                send_sem=send_sems.at[2 * phase + 1],
                recv_sem=recv_sems.at[2 * phase + 1],
                device_id=peer, device_id_type=pl.DeviceIdType.MESH)
            o_rdma.start()
            st_rdma.start()
            o_rdma.wait()
            st_rdma.wait()

            m_s = st_acc[0]
            l_s = st_acc[1]
            m_p = st_recv[phase, 0]
            l_p = st_recv[phase, 1]
            m_n = jnp.maximum(m_s, m_p)
            a = jnp.exp(m_s - m_n)
            bt = jnp.exp(m_p - m_n)
            st_acc[0] = m_n
            st_acc[1] = a * l_s + bt * l_p
            o_acc[...] = (a[..., None] * o_acc[...]
                          + bt[..., None] * o_recv[phase])

        o_ref[...] = o_acc[...] / st_acc[1][..., None]

        @functools.partial(pl.run_scoped, sem=pltpu.SemaphoreType.REGULAR)
        def _(sem):
            for peer in (y_peer, x_peer):
                pl.semaphore_signal(sem, inc=1, device_id=peer,
                                    device_id_type=pl.DeviceIdType.MESH)
            pl.semaphore_wait(sem, 2)

    return pl.pallas_call(
        body,
        out_shape=jax.ShapeDtypeStruct((B, SQ, H, D), jnp.float32),
        in_specs=[
            pl.BlockSpec(memory_space=pltpu.VMEM),
            pl.BlockSpec(memory_space=pl.ANY),
            pl.BlockSpec(memory_space=pl.ANY),
        ],
        out_specs=pl.BlockSpec(memory_space=pltpu.VMEM),
        scratch_shapes=[
            pltpu.VMEM((HALF, H, D), jnp.float32),
            pltpu.VMEM((HALF, H, D), jnp.float32),
            pltpu.VMEM((B, SQ, H, D), jnp.float32),
            pltpu.VMEM((2, B, SQ, H, D), jnp.float32),
            pltpu.VMEM((2, B, SQ, H), jnp.float32),
            pltpu.VMEM((2, 2, B, SQ, H), jnp.float32),
            pltpu.SemaphoreType.DMA((2,)),
            pltpu.SemaphoreType.DMA((4,)),
            pltpu.SemaphoreType.DMA((4,)),
        ],
        compiler_params=pltpu.CompilerParams(collective_id=0),
    )(Q, K, V)


# device time: 60900 ns/iter; 1.6162x vs baseline; 1.6162x over previous
import functools

import jax
import jax.numpy as jnp
from jax import lax
from jax.experimental import pallas as pl
from jax.experimental.pallas import tpu as pltpu

B, SQ, H, D = 8, 8, 16, 128
SKV_LOCAL = 1024
HALF = SKV_LOCAL // 2
SCALE = D ** -0.5


def kernel(Q, K, V):
    def body(q_ref, k_hbm, v_hbm, o_ref,
             k_buf, v_buf, o_acc, o_recv, st_acc, st_recv,
             copy_sems, send_sems, recv_sems):
        my_x = lax.axis_index("x")
        my_y = lax.axis_index("y")
        y_peer = (my_x, 1 - my_y)
        x_peer = (1 - my_x, my_y)

        barrier = pltpu.get_barrier_semaphore()
        for peer in (y_peer, x_peer):
            pl.semaphore_signal(barrier, inc=1, device_id=peer,
                                device_id_type=pl.DeviceIdType.MESH)
        pl.semaphore_wait(barrier, 2)

        row0 = my_x * HALF

        def dma_batch(b, slot):
            cps = []
            for h in range(H):
                cps.append(pltpu.make_async_copy(
                    k_hbm.at[b, pl.ds(row0, HALF), h],
                    k_buf.at[slot, h], copy_sems.at[slot, 0, h]))
                cps.append(pltpu.make_async_copy(
                    v_hbm.at[b, pl.ds(row0, HALF), h],
                    v_buf.at[slot, h], copy_sems.at[slot, 1, h]))
            for c in cps:
                c.start()
            return cps

        pending = dma_batch(0, 0)
        for b in range(B):
            slot = b % 2
            for c in pending:
                c.wait()
            if b + 1 < B:
                pending = dma_batch(b + 1, (b + 1) % 2)
            ms = []
            ls = []
            for h in range(H):
                q_h = q_ref[b, :, h, :]
                k_h = k_buf[slot, h]
                v_h = v_buf[slot, h]
                s = lax.dot_general(
                    q_h, k_h, (((1,), (1,)), ((), ())),
                    preferred_element_type=jnp.float32) * SCALE
                m = jnp.max(s, axis=-1)
                p = jnp.exp(s - m[:, None])
                l = jnp.sum(p, axis=-1)
                o_h = lax.dot_general(
                    p, v_h, (((1,), (0,)), ((), ())),
                    preferred_element_type=jnp.float32)
                o_acc[b, :, h, :] = o_h
                ms.append(m)
                ls.append(l)
            st_acc[0, b] = jnp.stack(ms, axis=1)
            st_acc[1, b] = jnp.stack(ls, axis=1)

        for phase, peer in enumerate((y_peer, x_peer)):
            o_rdma = pltpu.make_async_remote_copy(
                src_ref=o_acc, dst_ref=o_recv.at[phase],
                send_sem=send_sems.at[2 * phase],
                recv_sem=recv_sems.at[2 * phase],
                device_id=peer, device_id_type=pl.DeviceIdType.MESH)
            st_rdma = pltpu.make_async_remote_copy(
                src_ref=st_acc, dst_ref=st_recv.at[phase],
                send_sem=send_sems.at[2 * phase + 1],
                recv_sem=recv_sems.at[2 * phase + 1],
                device_id=peer, device_id_type=pl.DeviceIdType.MESH)
            o_rdma.start()
            st_rdma.start()
            o_rdma.wait()
            st_rdma.wait()

            m_s = st_acc[0]
            l_s = st_acc[1]
            m_p = st_recv[phase, 0]
            l_p = st_recv[phase, 1]
            m_n = jnp.maximum(m_s, m_p)
            a = jnp.exp(m_s - m_n)
            bt = jnp.exp(m_p - m_n)
            st_acc[0] = m_n
            st_acc[1] = a * l_s + bt * l_p
            o_acc[...] = (a[..., None] * o_acc[...]
                          + bt[..., None] * o_recv[phase])

        o_ref[...] = o_acc[...] / st_acc[1][..., None]

        @functools.partial(pl.run_scoped, sem=pltpu.SemaphoreType.REGULAR)
        def _(sem):
            for peer in (y_peer, x_peer):
                pl.semaphore_signal(sem, inc=1, device_id=peer,
                                    device_id_type=pl.DeviceIdType.MESH)
            pl.semaphore_wait(sem, 2)

    return pl.pallas_call(
        body,
        out_shape=jax.ShapeDtypeStruct((B, SQ, H, D), jnp.float32),
        in_specs=[
            pl.BlockSpec(memory_space=pltpu.VMEM),
            pl.BlockSpec(memory_space=pl.ANY),
            pl.BlockSpec(memory_space=pl.ANY),
        ],
        out_specs=pl.BlockSpec(memory_space=pltpu.VMEM),
        scratch_shapes=[
            pltpu.VMEM((2, H, HALF, D), jnp.float32),
            pltpu.VMEM((2, H, HALF, D), jnp.float32),
            pltpu.VMEM((B, SQ, H, D), jnp.float32),
            pltpu.VMEM((2, B, SQ, H, D), jnp.float32),
            pltpu.VMEM((2, B, SQ, H), jnp.float32),
            pltpu.VMEM((2, 2, B, SQ, H), jnp.float32),
            pltpu.SemaphoreType.DMA((2, 2, H)),
            pltpu.SemaphoreType.DMA((4,)),
            pltpu.SemaphoreType.DMA((4,)),
        ],
        compiler_params=pltpu.CompilerParams(
            collective_id=0, vmem_limit_bytes=64 * 1024 * 1024),
    )(Q, K, V)
